# baseline (speedup 1.0000x reference)
import dataclasses

import jax
import jax.numpy as jnp
from jax.experimental import pallas as pl
from jax.experimental.pallas import tpu as pltpu
from jax.experimental.pallas import tpu_sc as plsc

N = 10000
E = 320000
D = 128
N_PAD = 10240
NC, NS = 1, 16
NW = NC * NS
EPW = E // NW
BLK = 32
NBLK = EPW // BLK
LANES = 16
ZCOPIES = N_PAD // NS // BLK


def _gates_body(x_ref, wsgt_ref, bsg_ref, wdgt_ref, bdg_ref, tsrc_ref, edst_ref):
    x = x_ref[...]
    es = jnp.dot(x, wsgt_ref[...], preferred_element_type=jnp.float32)
    tsrc_ref[:, :D] = es + bsg_ref[...]
    tsrc_ref[:, D:] = x
    ed = jnp.dot(x, wdgt_ref[...], preferred_element_type=jnp.float32)
    edst_ref[...] = ed + bdg_ref[...]


def _gates(node_feats, wsgt, bsg, wdgt, bdg):
    nb = 2000
    return pl.pallas_call(
        _gates_body,
        grid=(N // nb,),
        in_specs=[
            pl.BlockSpec((nb, D), lambda i: (i, 0)),
            pl.BlockSpec((D, D), lambda i: (0, 0)),
            pl.BlockSpec((1, D), lambda i: (0, 0)),
            pl.BlockSpec((D, D), lambda i: (0, 0)),
            pl.BlockSpec((1, D), lambda i: (0, 0)),
        ],
        out_specs=[
            pl.BlockSpec((nb, 2 * D), lambda i: (i, 0)),
            pl.BlockSpec((nb, D), lambda i: (i, 0)),
        ],
        out_shape=[
            jax.ShapeDtypeStruct((N, 2 * D), jnp.float32),
            jax.ShapeDtypeStruct((N, D), jnp.float32),
        ],
    )(node_feats, wsgt, bsg, wdgt, bdg)


def _edge_gate_body(ef_ref, wegt_ref, beg_ref, o_ref):
    o = jnp.dot(ef_ref[...], wegt_ref[...], preferred_element_type=jnp.float32)
    o_ref[...] = o + beg_ref[...]


def _edge_gate(edge_feats, wegt, beg):
    eb = 4000
    return pl.pallas_call(
        _edge_gate_body,
        grid=(E // eb,),
        in_specs=[
            pl.BlockSpec((eb, D), lambda i: (i, 0)),
            pl.BlockSpec((D, D), lambda i: (0, 0)),
            pl.BlockSpec((1, D), lambda i: (0, 0)),
        ],
        out_specs=pl.BlockSpec((eb, D), lambda i: (i, 0)),
        out_shape=jax.ShapeDtypeStruct((E, D), jnp.float32),
    )(edge_feats, wegt, beg)


def _sc_body(tsrc_hbm, edst_hbm, eedge_hbm, src_hbm, dst_hbm,
             m_hbm, part_hbm, degp_hbm,
             sidx, didx, gsrc, gdst, gedge, msgbuf, deg,
             acc, sem1, sem2, sem3):
    c = jax.lax.axis_index("c")
    s = jax.lax.axis_index("s")
    wid = s * NC + c
    base0 = wid * EPW

    @pl.loop(0, BLK)
    def _(r):
        for j in range(D // LANES):
            msgbuf[r, pl.ds(j * LANES, LANES)] = jnp.zeros((LANES,), jnp.float32)

    @pl.loop(0, N_PAD // LANES)
    def _(i):
        deg[pl.ds(i * LANES, LANES)] = jnp.zeros((LANES,), jnp.float32)

    @pl.loop(0, ZCOPIES)
    def _(k):
        pltpu.sync_copy(msgbuf,
                        acc.at[pl.ds((s * ZCOPIES + k) * BLK, BLK), :])

    plsc.subcore_barrier()

    @pl.loop(0, NBLK)
    def _(it):
        base = base0 + it * BLK
        pltpu.sync_copy(src_hbm.at[pl.ds(base, BLK)], sidx)
        pltpu.sync_copy(dst_hbm.at[pl.ds(base, BLK)], didx)
        cp1 = pltpu.async_copy(tsrc_hbm.at[sidx], gsrc, sem1)
        cp2 = pltpu.async_copy(edst_hbm.at[didx], gdst, sem2)
        cp3 = pltpu.async_copy(eedge_hbm.at[pl.ds(base, BLK), :], gedge, sem3)
        cp1.wait()
        cp2.wait()
        cp3.wait()

        @pl.loop(0, BLK)
        def _(e):
            for j in range(D // LANES):
                sl = pl.ds(j * LANES, LANES)
                gs = gsrc[e, sl]
                nf = gsrc[e, pl.ds(D + j * LANES, LANES)]
                gd = gdst[e, sl]
                ge = gedge[e, sl]
                mm = gs + gd + ge
                gedge[e, sl] = mm
                sig = 1.0 / (1.0 + jnp.exp(-mm))
                msgbuf[e, sl] = nf * sig

        one0 = (jax.lax.iota(jnp.int32, LANES) == 0).astype(jnp.float32)

        @pl.loop(0, BLK // LANES)
        def _(g):
            dv = didx[pl.ds(g * LANES, LANES)]
            for k in range(LANES):
                d = dv[k]
                deg[pl.ds(d, LANES)] = deg[pl.ds(d, LANES)] + one0

        pltpu.sync_copy(gedge, m_hbm.at[pl.ds(base, BLK), :])
        pltpu.sync_copy(msgbuf, acc.at[didx], add=True)

    pltpu.sync_copy(deg, degp_hbm.at[wid])
    plsc.subcore_barrier()
    rows = N_PAD // NS
    pltpu.sync_copy(acc.at[pl.ds(s * rows, rows), :],
                    part_hbm.at[c, pl.ds(s * rows, rows), :])


def _sc_msgpass(tsrc, edst, eedge, src, dst):
    mesh = plsc.VectorSubcoreMesh(core_axis_name="c", subcore_axis_name="s",
                                  num_cores=NC, num_subcores=NS)
    cp = pltpu.CompilerParams()
    if "needs_layout_passes" in pltpu.CompilerParams.__dataclass_fields__:
        cp = dataclasses.replace(cp, needs_layout_passes=False)
    f = pl.kernel(
        _sc_body,
        compiler_params=cp,
        out_type=(
            jax.ShapeDtypeStruct((E, D), jnp.float32),
            jax.ShapeDtypeStruct((NC, N_PAD, D), jnp.float32),
            jax.ShapeDtypeStruct((NW, N_PAD), jnp.float32),
        ),
        mesh=mesh,
        scratch_types=[
            pltpu.VMEM((BLK,), jnp.int32),
            pltpu.VMEM((BLK,), jnp.int32),
            pltpu.VMEM((BLK, 2 * D), jnp.float32),
            pltpu.VMEM((BLK, D), jnp.float32),
            pltpu.VMEM((BLK, D), jnp.float32),
            pltpu.VMEM((BLK, D), jnp.float32),
            pltpu.VMEM((N_PAD,), jnp.float32),
            pltpu.VMEM_SHARED((N_PAD, D), jnp.float32),
            pltpu.SemaphoreType.DMA,
            pltpu.SemaphoreType.DMA,
            pltpu.SemaphoreType.DMA,
        ],
    )
    return f(tsrc, edst, eedge, src, dst)


def _final_body(x_ref, p_ref, dp_ref, wst_ref, wnt_ref, b_ref, o_ref):
    x = x_ref[...]
    p = p_ref[...]
    summed = jnp.sum(p, axis=0)
    deg = jnp.sum(dp_ref[...], axis=0)[:, None]
    hn = summed / jnp.maximum(deg, 1.0)
    out = x + jnp.dot(x, wst_ref[...], preferred_element_type=jnp.float32)
    out = out + jnp.dot(hn, wnt_ref[...], preferred_element_type=jnp.float32)
    o_ref[...] = out + b_ref[...]


def _final(node_feats, parts, degp, wst, wnt, b):
    nb = 2048
    return pl.pallas_call(
        _final_body,
        grid=(5,),
        in_specs=[
            pl.BlockSpec((nb, D), lambda i: (i, 0)),
            pl.BlockSpec((NC, nb, D), lambda i: (0, i, 0)),
            pl.BlockSpec((NW, nb), lambda i: (0, i)),
            pl.BlockSpec((D, D), lambda i: (0, 0)),
            pl.BlockSpec((D, D), lambda i: (0, 0)),
            pl.BlockSpec((1, D), lambda i: (0, 0)),
        ],
        out_specs=pl.BlockSpec((nb, D), lambda i: (i, 0)),
        out_shape=jax.ShapeDtypeStruct((N, D), jnp.float32),
    )(node_feats, parts, degp, wst, wnt, b)


def kernel(node_feats, edge_index, edge_feats,
           W_src_gate, b_src_gate, W_dst_gate, b_dst_gate,
           W_edge_gate, b_edge_gate, W_self, W_neigh, bias):
    src = edge_index[0].astype(jnp.int32)
    dst = edge_index[1].astype(jnp.int32)
    tsrc, edst = _gates(node_feats, W_src_gate.T, b_src_gate.reshape(1, D),
                        W_dst_gate.T, b_dst_gate.reshape(1, D))
    eedge = _edge_gate(edge_feats, W_edge_gate.T, b_edge_gate.reshape(1, D))
    m, parts, degp = _sc_msgpass(tsrc, edst, eedge, src, dst)
    rst = _final(node_feats, parts, degp, W_self.T, W_neigh.T,
                 bias.reshape(1, D))
    return (rst, m)
